# Initial kernel scaffold; baseline (speedup 1.0000x reference)
import jax
import jax.numpy as jnp
from jax.experimental import pallas as pl


def kernel(edge_index_title, edge_index_year, edge_index_cat, init_feat, W1, b1, W2, b2):
    raise NotImplementedError("write your pallas kernel here")


# speedup vs baseline: 10.7713x; 10.7713x over previous
import functools

import jax
import jax.numpy as jnp
from jax import lax
from jax.experimental import pallas as pl
from jax.experimental.pallas import tpu as pltpu
from jax.experimental.pallas import tpu_sc as plsc

N = 50000
D = 64
E = 800000
NP = 50176
HALF = 32
EPAD = 819200
ER = EPAD // 128
NC, NS = 2, 16
TROWS = NP // NS
SUP = 256
SUBS = SUP // 128
NSUP = EPAD // NS // SUP
DEG_TROWS = 3 * NP // NS

_f32 = jnp.float32


def _conv_body(ytab, srcr2, dstr, w, out, acc,
               ids0, idd0, rb0, ids1, idd1, rb1, zbuf, wrow,
               gsem0, gsem1, ssem0, ssem1):
    c = lax.axis_index("c")
    s = lax.axis_index("s")
    ids = (ids0, ids1)
    idd = (idd0, idd1)
    rb = (rb0, rb1)
    gsem = (gsem0, gsem1)
    ssem = (ssem0, ssem1)

    z16 = jnp.zeros((16,), _f32)

    @pl.loop(0, SUP, unroll=8)
    def _(r):
        zbuf[r, 0:16] = z16
        zbuf[r, 16:32] = z16

    r0 = s * TROWS
    for kk in range(12):
        pltpu.sync_copy(zbuf, acc.at[pl.ds(r0 + kk * SUP, SUP)])
    pltpu.sync_copy(zbuf.at[pl.ds(0, 64)], acc.at[pl.ds(r0 + 12 * SUP, 64)])
    plsc.subcore_barrier()

    base = s * (ER // NS)

    def load_and_fire(g, b):
        pltpu.sync_copy(srcr2.at[c, pl.ds(base + g * SUBS, SUBS)], ids[b])
        pltpu.sync_copy(dstr.at[pl.ds(base + g * SUBS, SUBS)], idd[b])
        for j in range(SUBS):
            pltpu.async_copy(ytab.at[ids[b].at[j]],
                             rb[b].at[pl.ds(j * 128, 128)], gsem[b])

    def drain_gather(b):
        for j in range(SUBS):
            pltpu.make_async_copy(ytab.at[ids[b].at[j]],
                                  rb[b].at[pl.ds(j * 128, 128)],
                                  gsem[b]).wait()

    def fire_scatter(b):
        for j in range(SUBS):
            pltpu.async_copy(rb[b].at[pl.ds(j * 128, 128)],
                             acc.at[idd[b].at[j]], ssem[b], add=True)

    def drain_scatter(b):
        for j in range(SUBS):
            pltpu.make_async_copy(rb[b].at[pl.ds(j * 128, 128)],
                                  acc.at[idd[b].at[j]], ssem[b]).wait()

    load_and_fire(0, 0)
    load_and_fire(1, 1)

    @pl.loop(0, NSUP // 2 - 1)
    def _(i):
        for b in range(2):
            g = 2 * i + b
            drain_gather(b)
            fire_scatter(b)
            drain_scatter(b)
            load_and_fire(g + 2, b)

    for b in range(2):
        drain_gather(b)
        fire_scatter(b)
        drain_scatter(b)

    plsc.subcore_barrier()

    for off, sz in tuple((kk * SUP, SUP) for kk in range(12)) + ((3072, 64),):
        pltpu.sync_copy(acc.at[pl.ds(r0 + off, sz)], rb0.at[pl.ds(0, sz)])
        pltpu.sync_copy(w.at[pl.ds(r0 + off, sz)], wrow.at[pl.ds(0, sz)])

        @pl.loop(0, sz // 16)
        def _(r16):
            wv = wrow[pl.ds(r16 * 16, 16)]
            for rr in range(16):
                sw = wv[rr]
                row = r16 * 16 + rr
                rb0[row, 0:16] = rb0[row, 0:16] * sw
                rb0[row, 16:32] = rb0[row, 16:32] * sw

        pltpu.sync_copy(rb0.at[pl.ds(0, sz)], out.at[c, pl.ds(r0 + off, sz)])


_conv_call = pl.kernel(
    _conv_body,
    out_type=jax.ShapeDtypeStruct((NC, NP, HALF), _f32),
    mesh=plsc.VectorSubcoreMesh(core_axis_name="c", subcore_axis_name="s",
                                num_cores=NC, num_subcores=NS),
    compiler_params=pltpu.CompilerParams(use_tc_tiling_on_sc=False),
    scratch_types=[
        pltpu.VMEM_SHARED((NP, HALF), _f32),
        pltpu.VMEM((SUBS, 128), jnp.int32),
        pltpu.VMEM((SUBS, 128), jnp.int32),
        pltpu.VMEM((SUP, HALF), _f32),
        pltpu.VMEM((SUBS, 128), jnp.int32),
        pltpu.VMEM((SUBS, 128), jnp.int32),
        pltpu.VMEM((SUP, HALF), _f32),
        pltpu.VMEM((SUP, HALF), _f32),
        pltpu.VMEM((SUP,), _f32),
        pltpu.SemaphoreType.DMA,
        pltpu.SemaphoreType.DMA,
        pltpu.SemaphoreType.DMA,
        pltpu.SemaphoreType.DMA,
    ],
)


def _fin_body(dt_ref, dy_ref, dc_ref, x_ref, z_ref, dinv2_ref):
    i = pl.program_id(0)
    x = x_ref[...]

    rows = lax.broadcasted_iota(jnp.int32, (512, 1), 0) + i * 512
    valid = rows < N

    zs = []
    d2s = []
    for dref in (dt_ref, dy_ref, dc_ref):
        degk = dref[0][:, 0:1]
        dinv = jnp.where((degk > 0.0) & valid, 1.0 / jnp.sqrt(degk), 0.0)
        zk = dinv * x
        zs.append(zk[:, 0:HALF])
        zs.append(zk[:, HALF:D])
        d2s.append(dinv * dinv)
    z_ref[...] = jnp.stack(zs, axis=0)
    dinv2_ref[...] = jnp.concatenate(
        d2s + [jnp.zeros((512, 5), _f32)], axis=1)


_fin_call = pl.pallas_call(
    _fin_body,
    grid=(NP // 512,),
    in_specs=[
        pl.BlockSpec((NC, 512, HALF), lambda i: (0, i, 0)),
        pl.BlockSpec((NC, 512, HALF), lambda i: (0, i, 0)),
        pl.BlockSpec((NC, 512, HALF), lambda i: (0, i, 0)),
        pl.BlockSpec((512, D), lambda i: (i, 0)),
    ],
    out_specs=[
        pl.BlockSpec((6, 512, HALF), lambda i: (0, i, 0)),
        pl.BlockSpec((512, 8), lambda i: (i, 0)),
    ],
    out_shape=[
        jax.ShapeDtypeStruct((6, NP, HALF), _f32),
        jax.ShapeDtypeStruct((NP, 8), _f32),
    ],
)


def _l2n(v):
    n = jnp.sqrt(jnp.sum(v * v, axis=1, keepdims=True))
    return v / jnp.maximum(n, 1e-12)


def _att_body(t_ref, y_ref, c_ref, w1_ref, b1_ref, w2_ref, b2_ref, o_ref):
    tn = _l2n(jnp.concatenate([t_ref[0], t_ref[1]], axis=1))
    yn = _l2n(jnp.concatenate([y_ref[0], y_ref[1]], axis=1))
    cn = _l2n(jnp.concatenate([c_ref[0], c_ref[1]], axis=1))

    xall = jnp.concatenate([tn, yn, cn], axis=1)
    h = jnp.maximum(
        jnp.dot(xall, w1_ref[...], preferred_element_type=_f32)
        + b1_ref[...], 0.0)
    lg = (jnp.dot(h, w2_ref[...], preferred_element_type=_f32)
          + b2_ref[...])

    lane = lax.broadcasted_iota(jnp.int32, (512, 128), 1)
    ml = jnp.where(lane < 3, lg, -jnp.inf)
    m = jnp.max(ml, axis=1, keepdims=True)
    e = jnp.exp(ml - m)
    att = e / jnp.sum(e, axis=1, keepdims=True)

    a = [jnp.sum(jnp.where(lane == k, att, 0.0), axis=1, keepdims=True)
         for k in range(3)]
    o_ref[...] = a[0] * tn + a[1] * yn + a[2] * cn


_att_call = pl.pallas_call(
    _att_body,
    grid=(NP // 512,),
    in_specs=[
        pl.BlockSpec((NC, 512, HALF), lambda i: (0, i, 0)),
        pl.BlockSpec((NC, 512, HALF), lambda i: (0, i, 0)),
        pl.BlockSpec((NC, 512, HALF), lambda i: (0, i, 0)),
        pl.BlockSpec((3 * D, 128), lambda i: (0, 0)),
        pl.BlockSpec((1, 128), lambda i: (0, 0)),
        pl.BlockSpec((128, 128), lambda i: (0, 0)),
        pl.BlockSpec((1, 128), lambda i: (0, 0)),
    ],
    out_specs=pl.BlockSpec((512, D), lambda i: (i, 0)),
    out_shape=jax.ShapeDtypeStruct((NP, D), _f32),
)


def _prep_edges(ei):
    src = ei[0].astype(jnp.int32)
    dst = ei[1].astype(jnp.int32)
    pad = jnp.full((EPAD - E,), N, jnp.int32)
    srcr = jnp.concatenate([src, pad]).reshape(ER, 128)
    dstr = jnp.concatenate([dst, pad]).reshape(ER, 128)
    srcr2 = jnp.stack([srcr, srcr + NP])
    return srcr2, dstr


@jax.jit
def kernel(edge_index_title, edge_index_year, edge_index_cat,
           init_feat, W1, b1, W2, b2):
    st2, dt = _prep_edges(edge_index_title)
    sy2, dy = _prep_edges(edge_index_year)
    sc2, dc = _prep_edges(edge_index_cat)

    ones_tab = jnp.ones((NC * NP, HALF), _f32)
    ones_w = jnp.ones((NP,), _f32)
    deg_t = _conv_call(ones_tab, st2, dt, ones_w)
    deg_y = _conv_call(ones_tab, sy2, dy, ones_w)
    deg_c = _conv_call(ones_tab, sc2, dc, ones_w)

    x_pad = jnp.concatenate(
        [init_feat.astype(_f32), jnp.zeros((NP - N, D), _f32)], axis=0)
    z, dinv2 = _fin_call(deg_t, deg_y, deg_c, x_pad)
    w_t2 = dinv2[:, 0]
    w_c2 = dinv2[:, 2]

    t1 = _conv_call(z[0:2].reshape(NC * NP, HALF), st2, dt, w_t2)
    traw = _conv_call(t1.reshape(NC * NP, HALF), st2, dt, ones_w)
    yraw = _conv_call(z[2:4].reshape(NC * NP, HALF), sy2, dy, ones_w)
    c1 = _conv_call(z[4:6].reshape(NC * NP, HALF), sc2, dc, w_c2)
    c2 = _conv_call(c1.reshape(NC * NP, HALF), sc2, dc, w_c2)
    craw = _conv_call(c2.reshape(NC * NP, HALF), sc2, dc, ones_w)

    out = _att_call(traw, yraw, craw, W1.astype(_f32),
                    b1.astype(_f32).reshape(1, 128),
                    jnp.pad(W2.astype(_f32), ((0, 0), (0, 125))),
                    jnp.pad(b2.astype(_f32), (0, 125)).reshape(1, 128))
    return out[:N]


# speedup vs baseline: 12.4807x; 1.1587x over previous
import jax
import jax.numpy as jnp
from jax import lax
from jax.experimental import pallas as pl
from jax.experimental.pallas import tpu as pltpu
from jax.experimental.pallas import tpu_sc as plsc

N = 50000
D = 64
E = 800000
NP = 50176
HALF = 32
EPAD = 823296
ER = EPAD // 128
NC, NS = 2, 16
TROWS = NP // NS
NSUP = ER // NS
NBUF = 6
DER = 6400

_f32 = jnp.float32
_mesh = plsc.VectorSubcoreMesh(core_axis_name="c", subcore_axis_name="s",
                               num_cores=NC, num_subcores=NS)
_sc_params = pltpu.CompilerParams(use_tc_tiling_on_sc=False)


def _deg_body(d0, d1, d2, ones_h, zeros_h, out, acc, ib0, ib1, onesb,
              ssem0, ssem1):
    c = lax.axis_index("c")
    s = lax.axis_index("s")
    ib = (ib0, ib1)
    ssem = (ssem0, ssem1)

    pltpu.sync_copy(ones_h, onesb)
    r0 = s * TROWS
    base = c * 3200 + s * 200

    for k, dref in enumerate((d0, d1, d2)):
        for kk in range(3):
            pltpu.sync_copy(zeros_h, acc.at[pl.ds(r0 + kk * 1024, 1024)])
        pltpu.sync_copy(zeros_h.at[pl.ds(0, 64)],
                        acc.at[pl.ds(r0 + 3072, 64)])
        plsc.subcore_barrier()

        def fire(g, b, _d=dref):
            pltpu.sync_copy(_d.at[pl.ds(base + g * 8, 8)], ib[b])
            for j in range(8):
                pltpu.async_copy(onesb, acc.at[ib[b].at[j]], ssem[b],
                                 add=True)

        def drain(b):
            for j in range(8):
                pltpu.make_async_copy(onesb, acc.at[ib[b].at[j]],
                                      ssem[b]).wait()

        fire(0, 0)
        fire(1, 1)

        @pl.loop(0, 11)
        def _(i):
            drain(0)
            fire(2 * i + 2, 0)
            drain(1)
            fire(2 * i + 3, 1)

        drain(0)
        fire(24, 0)
        drain(1)
        drain(0)
        plsc.subcore_barrier()

        for kk in range(3):
            pltpu.sync_copy(acc.at[pl.ds(r0 + kk * 1024, 1024)],
                            out.at[k, c, pl.ds(r0 + kk * 1024, 1024)])
        pltpu.sync_copy(acc.at[pl.ds(r0 + 3072, 64)],
                        out.at[k, c, pl.ds(r0 + 3072, 64)])
        plsc.subcore_barrier()


_deg_call = pl.kernel(
    _deg_body,
    out_type=jax.ShapeDtypeStruct((3, NC, NP, 16), _f32),
    mesh=_mesh,
    compiler_params=_sc_params,
    scratch_types=[
        pltpu.VMEM_SHARED((NP, 16), _f32),
        pltpu.VMEM((8, 128), jnp.int32),
        pltpu.VMEM((8, 128), jnp.int32),
        pltpu.VMEM((128, 16), _f32),
        pltpu.SemaphoreType.DMA,
        pltpu.SemaphoreType.DMA,
    ],
)


def _conv_body(ytab, srcr2, dstr, w, zeros_h, out, acc,
               ids0, idd0, rb0, ids1, idd1, rb1, ids2, idd2, rb2,
               ids3, idd3, rb3, ids4, idd4, rb4, ids5, idd5, rb5,
               wrow, gsem0, gsem1, gsem2, gsem3, gsem4, gsem5,
               ssem0, ssem1, ssem2, ssem3, ssem4, ssem5):
    c = lax.axis_index("c")
    s = lax.axis_index("s")
    ids = (ids0, ids1, ids2, ids3, ids4, ids5)
    idd = (idd0, idd1, idd2, idd3, idd4, idd5)
    rb = (rb0, rb1, rb2, rb3, rb4, rb5)
    gsem = (gsem0, gsem1, gsem2, gsem3, gsem4, gsem5)
    ssem = (ssem0, ssem1, ssem2, ssem3, ssem4, ssem5)

    r0 = s * TROWS
    for kk in range(3):
        pltpu.sync_copy(zeros_h, acc.at[pl.ds(r0 + kk * 1024, 1024)])
    pltpu.sync_copy(zeros_h.at[pl.ds(0, 64)], acc.at[pl.ds(r0 + 3072, 64)])
    plsc.subcore_barrier()

    base = s * NSUP

    def load_and_fire(g, b):
        pltpu.sync_copy(srcr2.at[c, pl.ds(base + g, 1)], ids[b])
        pltpu.sync_copy(dstr.at[pl.ds(base + g, 1)], idd[b])
        pltpu.async_copy(ytab.at[ids[b].at[0]], rb[b], gsem[b])

    def drain_gather(b):
        pltpu.make_async_copy(ytab.at[ids[b].at[0]], rb[b], gsem[b]).wait()

    def fire_scatter(b):
        pltpu.async_copy(rb[b], acc.at[idd[b].at[0]], ssem[b], add=True)

    def drain_scatter(b):
        pltpu.make_async_copy(rb[b], acc.at[idd[b].at[0]], ssem[b]).wait()

    for b in range(NBUF):
        load_and_fire(b, b)
    for u in range(3):
        drain_gather(u)
        fire_scatter(u)

    @pl.loop(0, (NSUP - NBUF) // NBUF)
    def _(i):
        for u in range(NBUF):
            g = NBUF * i + u + 3
            b = (u + 3) % NBUF
            drain_gather(b)
            fire_scatter(b)
            drain_scatter(u)
            load_and_fire(g + 3, u)

    for g in (NSUP - 3, NSUP - 2, NSUP - 1):
        drain_gather(g % NBUF)
        fire_scatter(g % NBUF)
    for g in range(NSUP - NBUF, NSUP):
        drain_scatter(g % NBUF)

    plsc.subcore_barrier()

    for off, sz in tuple((kk * 128, 128) for kk in range(24)) + ((3072, 64),):
        pltpu.sync_copy(acc.at[pl.ds(r0 + off, sz)], rb0.at[pl.ds(0, sz)])
        pltpu.sync_copy(w.at[pl.ds(r0 + off, sz)], wrow.at[pl.ds(0, sz)])

        @pl.loop(0, sz // 16)
        def _(r16):
            wv = wrow[pl.ds(r16 * 16, 16)]
            for rr in range(16):
                sw = wv[rr]
                row = r16 * 16 + rr
                rb0[row, 0:16] = rb0[row, 0:16] * sw
                rb0[row, 16:32] = rb0[row, 16:32] * sw

        pltpu.sync_copy(rb0.at[pl.ds(0, sz)], out.at[c, pl.ds(r0 + off, sz)])


_conv_call = pl.kernel(
    _conv_body,
    out_type=jax.ShapeDtypeStruct((NC, NP, HALF), _f32),
    mesh=_mesh,
    compiler_params=_sc_params,
    scratch_types=[pltpu.VMEM_SHARED((NP, HALF), _f32)]
    + [
        t
        for _ in range(NBUF)
        for t in (pltpu.VMEM((1, 128), jnp.int32),
                  pltpu.VMEM((1, 128), jnp.int32),
                  pltpu.VMEM((128, HALF), _f32))
    ]
    + [pltpu.VMEM((128,), _f32)]
    + [pltpu.SemaphoreType.DMA] * (2 * NBUF),
)


def _fin_body(dp_ref, x_ref, z_ref, dinv2_ref):
    i = pl.program_id(0)
    x = x_ref[...]

    rows = lax.broadcasted_iota(jnp.int32, (512, 1), 0) + i * 512
    valid = rows < N

    zs = []
    d2s = []
    for k in range(3):
        degk = dp_ref[k, 0][:, 0:1] + dp_ref[k, 1][:, 0:1]
        dinv = jnp.where((degk > 0.0) & valid, 1.0 / jnp.sqrt(degk), 0.0)
        zk = dinv * x
        zs.append(zk[:, 0:HALF])
        zs.append(zk[:, HALF:D])
        d2s.append(dinv * dinv)
    z_ref[...] = jnp.stack(zs, axis=0)
    dinv2_ref[...] = jnp.concatenate(
        d2s + [jnp.zeros((512, 5), _f32)], axis=1)


_fin_call = pl.pallas_call(
    _fin_body,
    grid=(NP // 512,),
    in_specs=[
        pl.BlockSpec((3, NC, 512, 16), lambda i: (0, 0, i, 0)),
        pl.BlockSpec((512, D), lambda i: (i, 0)),
    ],
    out_specs=[
        pl.BlockSpec((6, 512, HALF), lambda i: (0, i, 0)),
        pl.BlockSpec((512, 8), lambda i: (i, 0)),
    ],
    out_shape=[
        jax.ShapeDtypeStruct((6, NP, HALF), _f32),
        jax.ShapeDtypeStruct((NP, 8), _f32),
    ],
)


def _l2n(v):
    n = jnp.sqrt(jnp.sum(v * v, axis=1, keepdims=True))
    return v / jnp.maximum(n, 1e-12)


def _att_body(t_ref, y_ref, c_ref, w1_ref, b1_ref, w2_ref, b2_ref, o_ref):
    tn = _l2n(jnp.concatenate([t_ref[0], t_ref[1]], axis=1))
    yn = _l2n(jnp.concatenate([y_ref[0], y_ref[1]], axis=1))
    cn = _l2n(jnp.concatenate([c_ref[0], c_ref[1]], axis=1))

    xall = jnp.concatenate([tn, yn, cn], axis=1)
    h = jnp.maximum(
        jnp.dot(xall, w1_ref[...], preferred_element_type=_f32)
        + b1_ref[...], 0.0)
    lg = (jnp.dot(h, w2_ref[...], preferred_element_type=_f32)
          + b2_ref[...])

    lane = lax.broadcasted_iota(jnp.int32, (512, 128), 1)
    ml = jnp.where(lane < 3, lg, -jnp.inf)
    m = jnp.max(ml, axis=1, keepdims=True)
    e = jnp.exp(ml - m)
    att = e / jnp.sum(e, axis=1, keepdims=True)

    a = [jnp.sum(jnp.where(lane == k, att, 0.0), axis=1, keepdims=True)
         for k in range(3)]
    o_ref[...] = a[0] * tn + a[1] * yn + a[2] * cn


_att_call = pl.pallas_call(
    _att_body,
    grid=(NP // 512,),
    in_specs=[
        pl.BlockSpec((NC, 512, HALF), lambda i: (0, i, 0)),
        pl.BlockSpec((NC, 512, HALF), lambda i: (0, i, 0)),
        pl.BlockSpec((NC, 512, HALF), lambda i: (0, i, 0)),
        pl.BlockSpec((3 * D, 128), lambda i: (0, 0)),
        pl.BlockSpec((1, 128), lambda i: (0, 0)),
        pl.BlockSpec((128, 128), lambda i: (0, 0)),
        pl.BlockSpec((1, 128), lambda i: (0, 0)),
    ],
    out_specs=pl.BlockSpec((512, D), lambda i: (i, 0)),
    out_shape=jax.ShapeDtypeStruct((NP, D), _f32),
)


def _prep_edges(ei):
    src = ei[0].astype(jnp.int32)
    dst = ei[1].astype(jnp.int32)
    pad = jnp.full((EPAD - E,), N, jnp.int32)
    srcr = jnp.concatenate([src, pad]).reshape(ER, 128)
    dstr = jnp.concatenate([dst, pad]).reshape(ER, 128)
    srcr2 = jnp.stack([srcr, srcr + NP])
    return srcr2, dstr


@jax.jit
def kernel(edge_index_title, edge_index_year, edge_index_cat,
           init_feat, W1, b1, W2, b2):
    st2, dt = _prep_edges(edge_index_title)
    sy2, dy = _prep_edges(edge_index_year)
    sc2, dc = _prep_edges(edge_index_cat)

    zeros32 = jnp.zeros((1024, HALF), _f32)
    degparts = _deg_call(dt[:DER], dy[:DER], dc[:DER],
                         jnp.ones((128, 16), _f32),
                         jnp.zeros((1024, 16), _f32))

    x_pad = jnp.concatenate(
        [init_feat.astype(_f32), jnp.zeros((NP - N, D), _f32)], axis=0)
    z, dinv2 = _fin_call(degparts, x_pad)

    ones_w = jnp.ones((NP,), _f32)
    w_t2 = dinv2[:, 0]
    w_c2 = dinv2[:, 2]

    t1 = _conv_call(z[0:2].reshape(NC * NP, HALF), st2, dt, w_t2, zeros32)
    traw = _conv_call(t1.reshape(NC * NP, HALF), st2, dt, ones_w, zeros32)
    yraw = _conv_call(z[2:4].reshape(NC * NP, HALF), sy2, dy, ones_w, zeros32)
    c1 = _conv_call(z[4:6].reshape(NC * NP, HALF), sc2, dc, w_c2, zeros32)
    c2 = _conv_call(c1.reshape(NC * NP, HALF), sc2, dc, w_c2, zeros32)
    craw = _conv_call(c2.reshape(NC * NP, HALF), sc2, dc, ones_w, zeros32)

    out = _att_call(traw, yraw, craw, W1.astype(_f32),
                    b1.astype(_f32).reshape(1, 128),
                    jnp.pad(W2.astype(_f32), ((0, 0), (0, 125))),
                    jnp.pad(b2.astype(_f32), (0, 125)).reshape(1, 128))
    return out[:N]


# speedup vs baseline: 14.6986x; 1.1777x over previous
import jax
import jax.numpy as jnp
from jax import lax
from jax.experimental import pallas as pl
from jax.experimental.pallas import tpu as pltpu
from jax.experimental.pallas import tpu_sc as plsc

N = 50000
D = 64
E = 800000
NP = 50176
HALF = 32
EPAD = 819200
ER = EPAD // 128
NC, NS = 2, 16
TROWS = NP // NS
RPT = ER // NS
BLK = 16
NBLK = RPT // BLK
NBUF = 4

_f32 = jnp.float32
_mesh = plsc.VectorSubcoreMesh(core_axis_name="c", subcore_axis_name="s",
                               num_cores=NC, num_subcores=NS)
_sc_params = pltpu.CompilerParams(use_tc_tiling_on_sc=False)


def _deg_body(d0, d1, d2, ones_h, zeros_h, out, acc, ib0, ib1, onesb,
              ssem0, ssem1):
    c = lax.axis_index("c")
    s = lax.axis_index("s")
    ib = (ib0, ib1)
    ssem = (ssem0, ssem1)

    pltpu.sync_copy(ones_h, onesb)
    r0 = s * TROWS
    base = c * 3200 + s * 200

    for k, dref in enumerate((d0, d1, d2)):
        for kk in range(3):
            pltpu.sync_copy(zeros_h, acc.at[pl.ds(r0 + kk * 1024, 1024)])
        pltpu.sync_copy(zeros_h.at[pl.ds(0, 64)],
                        acc.at[pl.ds(r0 + 3072, 64)])
        plsc.subcore_barrier()

        def fire(g, b, _d=dref):
            pltpu.sync_copy(_d.at[pl.ds(base + g * 8, 8)], ib[b])
            for j in range(8):
                pltpu.async_copy(onesb, acc.at[ib[b].at[j]], ssem[b],
                                 add=True)

        def drain(b):
            for j in range(8):
                pltpu.make_async_copy(onesb, acc.at[ib[b].at[j]],
                                      ssem[b]).wait()

        fire(0, 0)
        fire(1, 1)

        @pl.loop(0, 11)
        def _(i):
            drain(0)
            fire(2 * i + 2, 0)
            drain(1)
            fire(2 * i + 3, 1)

        drain(0)
        fire(24, 0)
        drain(1)
        drain(0)
        plsc.subcore_barrier()

        for kk in range(3):
            pltpu.sync_copy(acc.at[pl.ds(r0 + kk * 1024, 1024)],
                            out.at[k, c, pl.ds(r0 + kk * 1024, 1024)])
        pltpu.sync_copy(acc.at[pl.ds(r0 + 3072, 64)],
                        out.at[k, c, pl.ds(r0 + 3072, 64)])
        plsc.subcore_barrier()


_deg_call = pl.kernel(
    _deg_body,
    out_type=jax.ShapeDtypeStruct((3, NC, NP, 16), _f32),
    mesh=_mesh,
    compiler_params=_sc_params,
    scratch_types=[
        pltpu.VMEM_SHARED((NP, 16), _f32),
        pltpu.VMEM((8, 128), jnp.int32),
        pltpu.VMEM((8, 128), jnp.int32),
        pltpu.VMEM((128, 16), _f32),
        pltpu.SemaphoreType.DMA,
        pltpu.SemaphoreType.DMA,
    ],
)


def _emit_conv(c, s, ytab, comb, w, out, acc, ixb, rb, gsem, ssem,
               zeros_h, wrow):
    r0 = s * TROWS
    base = s * RPT

    for kk in range(3):
        pltpu.sync_copy(zeros_h, acc.at[pl.ds(r0 + kk * 1024, 1024)])
    pltpu.sync_copy(zeros_h.at[pl.ds(0, 64)], acc.at[pl.ds(r0 + 3072, 64)])
    plsc.subcore_barrier()

    def load_idx(blk, p):
        pltpu.sync_copy(comb.at[c, pl.ds(base + blk * BLK, BLK)], ixb[p])

    def fire_gather(u, p, b):
        pltpu.async_copy(ytab.at[ixb[p].at[u, 0]], rb[b], gsem[b])

    def drain_gather(b):
        pltpu.make_async_copy(ytab.at[ixb[0].at[0, 0]], rb[b],
                              gsem[b]).wait()

    def fire_scatter(u, p, b):
        pltpu.async_copy(rb[b], acc.at[ixb[p].at[u, 1]], ssem[b], add=True)

    def drain_scatter(b):
        pltpu.make_async_copy(rb[b], acc.at[ixb[0].at[0, 1]],
                              ssem[b]).wait()

    def sup(u, p, fire=True, drain_sc=True, load_blk=None):
        b = u % NBUF
        drain_gather(b)
        fire_scatter(u, p, b)
        if drain_sc:
            drain_scatter((u - 2) % NBUF)
        if fire:
            if u + 2 < BLK:
                fire_gather(u + 2, p, (u + 2) % NBUF)
            else:
                fire_gather(u - 14, 1 - p, (u + 2) % NBUF)
        if load_blk is not None:
            load_idx(load_blk, 1 - p)

    load_idx(0, 0)
    load_idx(1, 1)
    fire_gather(0, 0, 0)
    fire_gather(1, 0, 1)
    sup(0, 0, drain_sc=False)
    sup(1, 0, drain_sc=False)
    for u in range(2, BLK):
        sup(u, 0)

    @pl.loop(0, 11)
    def _(i):
        blk1 = 2 * i + 1
        for u in range(BLK):
            sup(u, 1, load_blk=(blk1 + 1) if u == 4 else None)
        blk2 = 2 * i + 2
        for u in range(BLK):
            sup(u, 0, load_blk=(blk2 + 1) if u == 4 else None)

    for u in range(BLK):
        sup(u, 1, load_blk=24 if u == 4 else None)
    for u in range(BLK - 2):
        sup(u, 0)
    sup(BLK - 2, 0, fire=False)
    sup(BLK - 1, 0, fire=False)
    drain_scatter(2)
    drain_scatter(3)

    plsc.subcore_barrier()

    def scale_rows(nr):
        @pl.loop(0, nr // 16)
        def _(r16):
            wv = wrow[pl.ds(r16 * 16, 16)]
            for rr in range(16):
                sw = wv[rr]
                row = r16 * 16 + rr
                rb[0][row, 0:16] = rb[0][row, 0:16] * sw
                rb[0][row, 16:32] = rb[0][row, 16:32] * sw

    @pl.loop(0, 24)
    def _(kk):
        off = r0 + kk * 128
        pltpu.sync_copy(acc.at[pl.ds(off, 128)], rb[0])
        pltpu.sync_copy(w.at[pl.ds(off, 128)], wrow)
        scale_rows(128)
        pltpu.sync_copy(rb[0], out.at[pl.ds(c * NP + off, 128)])

    off = r0 + 3072
    pltpu.sync_copy(acc.at[pl.ds(off, 64)], rb[0].at[pl.ds(0, 64)])
    pltpu.sync_copy(w.at[pl.ds(off, 64)], wrow.at[pl.ds(0, 64)])
    scale_rows(64)
    pltpu.sync_copy(rb[0].at[pl.ds(0, 64)],
                    out.at[pl.ds(c * NP + off, 64)])
    plsc.subcore_barrier()


def _conv6_body(zt, zy, zc, cmb_t, cmb_y, cmb_c, w_t2, w_c2, ones_w,
                zeros_h, t1b, traw, yraw, c1b, c2b, craw, acc,
                ixb0, ixb1, rb0, rb1, rb2, rb3, wrow,
                gsem0, gsem1, gsem2, gsem3, ssem0, ssem1, ssem2, ssem3):
    c = lax.axis_index("c")
    s = lax.axis_index("s")
    ixb = (ixb0, ixb1)
    rb = (rb0, rb1, rb2, rb3)
    gsem = (gsem0, gsem1, gsem2, gsem3)
    ssem = (ssem0, ssem1, ssem2, ssem3)

    jobs = (
        (zt, cmb_t, w_t2, t1b),
        (t1b, cmb_t, ones_w, traw),
        (zy, cmb_y, ones_w, yraw),
        (zc, cmb_c, w_c2, c1b),
        (c1b, cmb_c, w_c2, c2b),
        (c2b, cmb_c, ones_w, craw),
    )
    for ytab, comb, w, out in jobs:
        _emit_conv(c, s, ytab, comb, w, out, acc, ixb, rb, gsem, ssem,
                   zeros_h, wrow)


_conv6_call = pl.kernel(
    _conv6_body,
    out_type=[jax.ShapeDtypeStruct((NC * NP, HALF), _f32)] * 6,
    mesh=_mesh,
    compiler_params=_sc_params,
    scratch_types=[pltpu.VMEM_SHARED((NP, HALF), _f32)]
    + [pltpu.VMEM((BLK, 2, 128), jnp.int32)] * 2
    + [pltpu.VMEM((128, HALF), _f32)] * NBUF
    + [pltpu.VMEM((128,), _f32)]
    + [pltpu.SemaphoreType.DMA] * (2 * NBUF),
)


def _fin_body(dp_ref, x_ref, z_ref, dinv2_ref):
    i = pl.program_id(0)
    x = x_ref[...]

    rows = lax.broadcasted_iota(jnp.int32, (512, 1), 0) + i * 512
    valid = rows < N

    zs = []
    d2s = []
    for k in range(3):
        degk = dp_ref[k, 0][:, 0:1] + dp_ref[k, 1][:, 0:1]
        dinv = jnp.where((degk > 0.0) & valid, 1.0 / jnp.sqrt(degk), 0.0)
        zk = dinv * x
        zs.append(zk[:, 0:HALF])
        zs.append(zk[:, HALF:D])
        d2s.append(dinv * dinv)
    z_ref[...] = jnp.stack(zs, axis=0)
    dinv2_ref[...] = jnp.concatenate(
        d2s + [jnp.zeros((512, 5), _f32)], axis=1)


_fin_call = pl.pallas_call(
    _fin_body,
    grid=(NP // 512,),
    in_specs=[
        pl.BlockSpec((3, NC, 512, 16), lambda i: (0, 0, i, 0)),
        pl.BlockSpec((512, D), lambda i: (i, 0)),
    ],
    out_specs=[
        pl.BlockSpec((6, 512, HALF), lambda i: (0, i, 0)),
        pl.BlockSpec((512, 8), lambda i: (i, 0)),
    ],
    out_shape=[
        jax.ShapeDtypeStruct((6, NP, HALF), _f32),
        jax.ShapeDtypeStruct((NP, 8), _f32),
    ],
)


def _l2n(v):
    n = jnp.sqrt(jnp.sum(v * v, axis=1, keepdims=True))
    return v / jnp.maximum(n, 1e-12)


def _att_body(t_ref, y_ref, c_ref, w1_ref, b1_ref, w2_ref, b2_ref, o_ref):
    tn = _l2n(jnp.concatenate([t_ref[0], t_ref[1]], axis=1))
    yn = _l2n(jnp.concatenate([y_ref[0], y_ref[1]], axis=1))
    cn = _l2n(jnp.concatenate([c_ref[0], c_ref[1]], axis=1))

    xall = jnp.concatenate([tn, yn, cn], axis=1)
    h = jnp.maximum(
        jnp.dot(xall, w1_ref[...], preferred_element_type=_f32)
        + b1_ref[...], 0.0)
    lg = (jnp.dot(h, w2_ref[...], preferred_element_type=_f32)
          + b2_ref[...])

    lane = lax.broadcasted_iota(jnp.int32, (512, 128), 1)
    ml = jnp.where(lane < 3, lg, -jnp.inf)
    m = jnp.max(ml, axis=1, keepdims=True)
    e = jnp.exp(ml - m)
    att = e / jnp.sum(e, axis=1, keepdims=True)

    a = [jnp.sum(jnp.where(lane == k, att, 0.0), axis=1, keepdims=True)
         for k in range(3)]
    o_ref[...] = a[0] * tn + a[1] * yn + a[2] * cn


_att_call = pl.pallas_call(
    _att_body,
    grid=(NP // 512,),
    in_specs=[
        pl.BlockSpec((NC, 512, HALF), lambda i: (0, i, 0)),
        pl.BlockSpec((NC, 512, HALF), lambda i: (0, i, 0)),
        pl.BlockSpec((NC, 512, HALF), lambda i: (0, i, 0)),
        pl.BlockSpec((3 * D, 128), lambda i: (0, 0)),
        pl.BlockSpec((1, 128), lambda i: (0, 0)),
        pl.BlockSpec((128, 128), lambda i: (0, 0)),
        pl.BlockSpec((1, 128), lambda i: (0, 0)),
    ],
    out_specs=pl.BlockSpec((512, D), lambda i: (i, 0)),
    out_shape=jax.ShapeDtypeStruct((NP, D), _f32),
)


def _prep_edges(ei):
    src = ei[0].astype(jnp.int32)
    dst = ei[1].astype(jnp.int32)
    pad = jnp.full((EPAD - E,), N, jnp.int32)
    srcr = jnp.concatenate([src, pad]).reshape(ER, 128)
    dstr = jnp.concatenate([dst, pad]).reshape(ER, 128)
    comb = jnp.stack([jnp.stack([srcr, dstr], axis=1),
                      jnp.stack([srcr + NP, dstr], axis=1)])
    return comb, dstr


@jax.jit
def kernel(edge_index_title, edge_index_year, edge_index_cat,
           init_feat, W1, b1, W2, b2):
    cmb_t, dt = _prep_edges(edge_index_title)
    cmb_y, dy = _prep_edges(edge_index_year)
    cmb_c, dc = _prep_edges(edge_index_cat)

    degparts = _deg_call(dt, dy, dc,
                         jnp.ones((128, 16), _f32),
                         jnp.zeros((1024, 16), _f32))

    x_pad = jnp.concatenate(
        [init_feat.astype(_f32), jnp.zeros((NP - N, D), _f32)], axis=0)
    z, dinv2 = _fin_call(degparts, x_pad)

    ones_w = jnp.ones((NP,), _f32)

    outs = _conv6_call(z[0:2].reshape(NC * NP, HALF),
                       z[2:4].reshape(NC * NP, HALF),
                       z[4:6].reshape(NC * NP, HALF),
                       cmb_t, cmb_y, cmb_c,
                       dinv2[:, 0], dinv2[:, 2], ones_w,
                       jnp.zeros((1024, HALF), _f32))
    traw = outs[1].reshape(NC, NP, HALF)
    yraw = outs[2].reshape(NC, NP, HALF)
    craw = outs[5].reshape(NC, NP, HALF)

    out = _att_call(traw, yraw, craw, W1.astype(_f32),
                    b1.astype(_f32).reshape(1, 128),
                    jnp.pad(W2.astype(_f32), ((0, 0), (0, 125))),
                    jnp.pad(b2.astype(_f32), (0, 125)).reshape(1, 128))
    return out[:N]


# speedup vs baseline: 14.7358x; 1.0025x over previous
import jax
import jax.numpy as jnp
from jax import lax
from jax.experimental import pallas as pl
from jax.experimental.pallas import tpu as pltpu
from jax.experimental.pallas import tpu_sc as plsc

N = 50000
D = 64
E = 800000
NP = 50176
HALF = 32
EPAD = 819200
ER = EPAD // 128
NC, NS = 2, 16
TROWS = NP // NS
RPT = ER // NS
BLK = 16
NBLK = RPT // BLK
NBUF = 4

_f32 = jnp.float32
_mesh = plsc.VectorSubcoreMesh(core_axis_name="c", subcore_axis_name="s",
                               num_cores=NC, num_subcores=NS)
_sc_params = pltpu.CompilerParams(use_tc_tiling_on_sc=False)


def _deg_body(d0, d1, d2, ones_h, zeros_h, out, acc, ib0, ib1, onesb,
              ssem0, ssem1):
    c = lax.axis_index("c")
    s = lax.axis_index("s")
    ib = (ib0, ib1)
    ssem = (ssem0, ssem1)

    pltpu.sync_copy(ones_h, onesb)
    r0 = s * TROWS
    base = c * 3200 + s * 200

    for k, dref in enumerate((d0, d1, d2)):
        for kk in range(3):
            pltpu.sync_copy(zeros_h, acc.at[pl.ds(r0 + kk * 1024, 1024)])
        pltpu.sync_copy(zeros_h.at[pl.ds(0, 64)],
                        acc.at[pl.ds(r0 + 3072, 64)])
        plsc.subcore_barrier()

        def fire(g, b, _d=dref):
            pltpu.sync_copy(_d.at[pl.ds(base + g * 8, 8)], ib[b])
            for j in range(8):
                pltpu.async_copy(onesb, acc.at[ib[b].at[j]], ssem[b],
                                 add=True)

        def drain(b):
            for j in range(8):
                pltpu.make_async_copy(onesb, acc.at[ib[b].at[j]],
                                      ssem[b]).wait()

        fire(0, 0)
        fire(1, 1)

        @pl.loop(0, 11)
        def _(i):
            drain(0)
            fire(2 * i + 2, 0)
            drain(1)
            fire(2 * i + 3, 1)

        drain(0)
        fire(24, 0)
        drain(1)
        drain(0)
        plsc.subcore_barrier()

        for kk in range(3):
            pltpu.sync_copy(acc.at[pl.ds(r0 + kk * 1024, 1024)],
                            out.at[k, c, pl.ds(r0 + kk * 1024, 1024)])
        pltpu.sync_copy(acc.at[pl.ds(r0 + 3072, 64)],
                        out.at[k, c, pl.ds(r0 + 3072, 64)])
        plsc.subcore_barrier()


_deg_call = pl.kernel(
    _deg_body,
    out_type=jax.ShapeDtypeStruct((3, NC, NP, 16), _f32),
    mesh=_mesh,
    compiler_params=_sc_params,
    scratch_types=[
        pltpu.VMEM_SHARED((NP, 16), _f32),
        pltpu.VMEM((8, 128), jnp.int32),
        pltpu.VMEM((8, 128), jnp.int32),
        pltpu.VMEM((128, 16), _f32),
        pltpu.SemaphoreType.DMA,
        pltpu.SemaphoreType.DMA,
    ],
)


def _emit_conv(c, s, ytab, comb, w, out, acc, ixb, rb, gsem, ssem, isem,
               zeros_h, wrow):
    r0 = s * TROWS
    base = s * RPT

    for kk in range(3):
        pltpu.sync_copy(zeros_h, acc.at[pl.ds(r0 + kk * 1024, 1024)])
    pltpu.sync_copy(zeros_h.at[pl.ds(0, 64)], acc.at[pl.ds(r0 + 3072, 64)])
    plsc.subcore_barrier()

    def load_idx(blk, p):
        pltpu.sync_copy(comb.at[c, pl.ds(base + blk * BLK, BLK)], ixb[p])

    def load_idx_async(blk, p):
        pltpu.async_copy(comb.at[c, pl.ds(base + blk * BLK, BLK)], ixb[p],
                         isem[p])

    def drain_idx(p):
        pltpu.make_async_copy(comb.at[c, pl.ds(base, BLK)], ixb[p],
                              isem[p]).wait()

    def fire_gather(u, p, b):
        pltpu.async_copy(ytab.at[ixb[p].at[u, 0]], rb[b], gsem[b])

    def drain_gather(b):
        pltpu.make_async_copy(ytab.at[ixb[0].at[0, 0]], rb[b],
                              gsem[b]).wait()

    def fire_scatter(u, p, b):
        pltpu.async_copy(rb[b], acc.at[ixb[p].at[u, 1]], ssem[b], add=True)

    def drain_scatter(b):
        pltpu.make_async_copy(rb[b], acc.at[ixb[0].at[0, 1]],
                              ssem[b]).wait()

    def sup(u, p, fire=True, drain_sc=True, load_blk=None, drain_ix=False):
        b = u % NBUF
        drain_gather(b)
        fire_scatter(u, p, b)
        if drain_sc:
            drain_scatter((u - 2) % NBUF)
        if fire:
            if u + 2 < BLK:
                fire_gather(u + 2, p, (u + 2) % NBUF)
            else:
                if drain_ix:
                    drain_idx(1 - p)
                fire_gather(u - 14, 1 - p, (u + 2) % NBUF)
        if load_blk is not None:
            load_idx_async(load_blk, 1 - p)

    load_idx(0, 0)
    load_idx(1, 1)
    fire_gather(0, 0, 0)
    fire_gather(1, 0, 1)
    sup(0, 0, drain_sc=False)
    sup(1, 0, drain_sc=False)
    for u in range(2, BLK):
        sup(u, 0)

    @pl.loop(0, 11)
    def _(i):
        blk1 = 2 * i + 1
        for u in range(BLK):
            sup(u, 1, load_blk=(blk1 + 1) if u == 4 else None,
                drain_ix=(u == 14))
        blk2 = 2 * i + 2
        for u in range(BLK):
            sup(u, 0, load_blk=(blk2 + 1) if u == 4 else None,
                drain_ix=(u == 14))

    for u in range(BLK):
        sup(u, 1, load_blk=24 if u == 4 else None, drain_ix=(u == 14))
    for u in range(BLK - 2):
        sup(u, 0)
    sup(BLK - 2, 0, fire=False)
    sup(BLK - 1, 0, fire=False)
    drain_scatter(2)
    drain_scatter(3)

    plsc.subcore_barrier()

    def scale_rows(nr):
        @pl.loop(0, nr // 16)
        def _(r16):
            wv = wrow[pl.ds(r16 * 16, 16)]
            for rr in range(16):
                sw = wv[rr]
                row = r16 * 16 + rr
                rb[0][row, 0:16] = rb[0][row, 0:16] * sw
                rb[0][row, 16:32] = rb[0][row, 16:32] * sw

    @pl.loop(0, 24)
    def _(kk):
        off = r0 + kk * 128
        pltpu.sync_copy(acc.at[pl.ds(off, 128)], rb[0])
        pltpu.sync_copy(w.at[pl.ds(off, 128)], wrow)
        scale_rows(128)
        pltpu.sync_copy(rb[0], out.at[pl.ds(c * NP + off, 128)])

    off = r0 + 3072
    pltpu.sync_copy(acc.at[pl.ds(off, 64)], rb[0].at[pl.ds(0, 64)])
    pltpu.sync_copy(w.at[pl.ds(off, 64)], wrow.at[pl.ds(0, 64)])
    scale_rows(64)
    pltpu.sync_copy(rb[0].at[pl.ds(0, 64)],
                    out.at[pl.ds(c * NP + off, 64)])
    plsc.subcore_barrier()


def _conv6_body(zt, zy, zc, cmb_t, cmb_y, cmb_c, w_t2, w_c2, ones_w,
                zeros_h, t1b, traw, yraw, c1b, c2b, craw, acc,
                ixb0, ixb1, rb0, rb1, rb2, rb3, wrow,
                gsem0, gsem1, gsem2, gsem3, ssem0, ssem1, ssem2, ssem3,
                isem0, isem1):
    c = lax.axis_index("c")
    s = lax.axis_index("s")
    ixb = (ixb0, ixb1)
    rb = (rb0, rb1, rb2, rb3)
    gsem = (gsem0, gsem1, gsem2, gsem3)
    ssem = (ssem0, ssem1, ssem2, ssem3)
    isem = (isem0, isem1)

    jobs = (
        (zt, cmb_t, w_t2, t1b),
        (t1b, cmb_t, ones_w, traw),
        (zy, cmb_y, ones_w, yraw),
        (zc, cmb_c, w_c2, c1b),
        (c1b, cmb_c, w_c2, c2b),
        (c2b, cmb_c, ones_w, craw),
    )
    for ytab, comb, w, out in jobs:
        _emit_conv(c, s, ytab, comb, w, out, acc, ixb, rb, gsem, ssem,
                   isem, zeros_h, wrow)


_conv6_call = pl.kernel(
    _conv6_body,
    out_type=[jax.ShapeDtypeStruct((NC * NP, HALF), _f32)] * 6,
    mesh=_mesh,
    compiler_params=_sc_params,
    scratch_types=[pltpu.VMEM_SHARED((NP, HALF), _f32)]
    + [pltpu.VMEM((BLK, 2, 128), jnp.int32)] * 2
    + [pltpu.VMEM((128, HALF), _f32)] * NBUF
    + [pltpu.VMEM((128,), _f32)]
    + [pltpu.SemaphoreType.DMA] * (2 * NBUF + 2),
)


def _fin_body(dp_ref, x_ref, z_ref, dinv2_ref):
    i = pl.program_id(0)
    x = x_ref[...]

    rows = lax.broadcasted_iota(jnp.int32, (512, 1), 0) + i * 512
    valid = rows < N

    zs = []
    d2s = []
    for k in range(3):
        degk = dp_ref[k, 0][:, 0:1] + dp_ref[k, 1][:, 0:1]
        dinv = jnp.where((degk > 0.0) & valid, 1.0 / jnp.sqrt(degk), 0.0)
        zk = dinv * x
        zs.append(zk[:, 0:HALF])
        zs.append(zk[:, HALF:D])
        d2s.append(dinv * dinv)
    z_ref[...] = jnp.stack(zs, axis=0)
    dinv2_ref[...] = jnp.concatenate(
        d2s + [jnp.zeros((512, 5), _f32)], axis=1)


_fin_call = pl.pallas_call(
    _fin_body,
    grid=(NP // 512,),
    in_specs=[
        pl.BlockSpec((3, NC, 512, 16), lambda i: (0, 0, i, 0)),
        pl.BlockSpec((512, D), lambda i: (i, 0)),
    ],
    out_specs=[
        pl.BlockSpec((6, 512, HALF), lambda i: (0, i, 0)),
        pl.BlockSpec((512, 8), lambda i: (i, 0)),
    ],
    out_shape=[
        jax.ShapeDtypeStruct((6, NP, HALF), _f32),
        jax.ShapeDtypeStruct((NP, 8), _f32),
    ],
)


def _l2n(v):
    n = jnp.sqrt(jnp.sum(v * v, axis=1, keepdims=True))
    return v / jnp.maximum(n, 1e-12)


def _att_body(t_ref, y_ref, c_ref, w1_ref, b1_ref, w2_ref, b2_ref, o_ref):
    tn = _l2n(jnp.concatenate([t_ref[0], t_ref[1]], axis=1))
    yn = _l2n(jnp.concatenate([y_ref[0], y_ref[1]], axis=1))
    cn = _l2n(jnp.concatenate([c_ref[0], c_ref[1]], axis=1))

    xall = jnp.concatenate([tn, yn, cn], axis=1)
    h = jnp.maximum(
        jnp.dot(xall, w1_ref[...], preferred_element_type=_f32)
        + b1_ref[...], 0.0)
    lg = (jnp.dot(h, w2_ref[...], preferred_element_type=_f32)
          + b2_ref[...])

    lane = lax.broadcasted_iota(jnp.int32, (512, 128), 1)
    ml = jnp.where(lane < 3, lg, -jnp.inf)
    m = jnp.max(ml, axis=1, keepdims=True)
    e = jnp.exp(ml - m)
    att = e / jnp.sum(e, axis=1, keepdims=True)

    a = [jnp.sum(jnp.where(lane == k, att, 0.0), axis=1, keepdims=True)
         for k in range(3)]
    o_ref[...] = a[0] * tn + a[1] * yn + a[2] * cn


_att_call = pl.pallas_call(
    _att_body,
    grid=(NP // 512,),
    in_specs=[
        pl.BlockSpec((NC, 512, HALF), lambda i: (0, i, 0)),
        pl.BlockSpec((NC, 512, HALF), lambda i: (0, i, 0)),
        pl.BlockSpec((NC, 512, HALF), lambda i: (0, i, 0)),
        pl.BlockSpec((3 * D, 128), lambda i: (0, 0)),
        pl.BlockSpec((1, 128), lambda i: (0, 0)),
        pl.BlockSpec((128, 128), lambda i: (0, 0)),
        pl.BlockSpec((1, 128), lambda i: (0, 0)),
    ],
    out_specs=pl.BlockSpec((512, D), lambda i: (i, 0)),
    out_shape=jax.ShapeDtypeStruct((NP, D), _f32),
)


def _prep_edges(ei):
    src = ei[0].astype(jnp.int32)
    dst = ei[1].astype(jnp.int32)
    pad = jnp.full((EPAD - E,), N, jnp.int32)
    srcr = jnp.concatenate([src, pad]).reshape(ER, 128)
    dstr = jnp.concatenate([dst, pad]).reshape(ER, 128)
    comb = jnp.stack([jnp.stack([srcr, dstr], axis=1),
                      jnp.stack([srcr + NP, dstr], axis=1)])
    return comb, dstr


@jax.jit
def kernel(edge_index_title, edge_index_year, edge_index_cat,
           init_feat, W1, b1, W2, b2):
    cmb_t, dt = _prep_edges(edge_index_title)
    cmb_y, dy = _prep_edges(edge_index_year)
    cmb_c, dc = _prep_edges(edge_index_cat)

    degparts = _deg_call(dt, dy, dc,
                         jnp.ones((128, 16), _f32),
                         jnp.zeros((1024, 16), _f32))

    x_pad = jnp.concatenate(
        [init_feat.astype(_f32), jnp.zeros((NP - N, D), _f32)], axis=0)
    z, dinv2 = _fin_call(degparts, x_pad)

    ones_w = jnp.ones((NP,), _f32)

    outs = _conv6_call(z[0:2].reshape(NC * NP, HALF),
                       z[2:4].reshape(NC * NP, HALF),
                       z[4:6].reshape(NC * NP, HALF),
                       cmb_t, cmb_y, cmb_c,
                       dinv2[:, 0], dinv2[:, 2], ones_w,
                       jnp.zeros((1024, HALF), _f32))
    traw = outs[1].reshape(NC, NP, HALF)
    yraw = outs[2].reshape(NC, NP, HALF)
    craw = outs[5].reshape(NC, NP, HALF)

    out = _att_call(traw, yraw, craw, W1.astype(_f32),
                    b1.astype(_f32).reshape(1, 128),
                    jnp.pad(W2.astype(_f32), ((0, 0), (0, 125))),
                    jnp.pad(b2.astype(_f32), (0, 125)).reshape(1, 128))
    return out[:N]
